# baseline (speedup 1.0000x reference)
import functools

import jax
import jax.numpy as jnp
from jax import lax
from jax.experimental import pallas as pl
from jax.experimental.pallas import tpu as pltpu
from jax.experimental.pallas import tpu_sc as plsc

_NC, _NS = 2, 16
_NW = _NC * _NS
_C = 16


def _gather_call(S_batch, S_seq, D, G):
    mesh = plsc.VectorSubcoreMesh(core_axis_name="c", subcore_axis_name="s")
    per_w = G * _C
    w_per_row = S_seq // per_w

    @functools.partial(
        pl.kernel,
        out_type=jax.ShapeDtypeStruct((S_batch, S_seq, D), jnp.float32),
        mesh=mesh,
        scratch_types=[
            pltpu.VMEM((per_w,), jnp.int32),
            pltpu.VMEM((_C, D), jnp.float32),
            pltpu.VMEM((_C, D), jnp.float32),
            pltpu.VMEM((_C, D), jnp.float32),
            pltpu.VMEM((_C, D), jnp.float32),
            pltpu.VMEM((_C, D), jnp.float32),
            pltpu.VMEM((_C, D), jnp.float32),
            pltpu.SemaphoreType.DMA,
            pltpu.SemaphoreType.DMA,
        ],
    )
    def k(idx_hbm, table_hbm, out_hbm, idx_v, rows0, rows1, rows2, rows3, rows4, rows5, gsem, ssem):
        wid = lax.axis_index("s") * _NC + lax.axis_index("c")
        b = wid // w_per_row
        col = (wid % w_per_row) * per_w
        pltpu.sync_copy(idx_hbm.at[b, pl.ds(col, per_w)], idx_v)
        bufs = (rows0, rows1, rows2, rows3, rows4, rows5)
        nb = len(bufs)

        gathers = [None] * G
        scats = [None] * G
        for g in range(nb - 1):
            gathers[g] = pltpu.async_copy(
                table_hbm.at[idx_v.at[pl.ds(g * _C, _C)]], bufs[g], gsem)
        for g in range(G):
            gathers[g].wait()
            if g + nb - 1 < G:
                if g >= 1:
                    scats[g - 1].wait()
                gathers[g + nb - 1] = pltpu.async_copy(
                    table_hbm.at[idx_v.at[pl.ds((g + nb - 1) * _C, _C)]],
                    bufs[(g + nb - 1) % nb], gsem)
            scats[g] = pltpu.async_copy(
                bufs[g % nb], out_hbm.at[b, pl.ds(col + g * _C, _C)], ssem)
        for g in range(G - nb, G):
            if g >= 0:
                scats[g].wait()

    return k


def kernel(positions, pos_emb):
    S_batch, S_seq = positions.shape
    D = pos_emb.shape[1]
    G = positions.size // (_NW * _C)
    idx = positions.astype(jnp.int32)
    return _gather_call(S_batch, S_seq, D, G)(idx, pos_emb)


# speedup vs baseline: 1.0130x; 1.0130x over previous
import functools

import jax
import jax.numpy as jnp
from jax import lax
from jax.experimental import pallas as pl
from jax.experimental.pallas import tpu as pltpu
from jax.experimental.pallas import tpu_sc as plsc

_NC, _NS = 2, 16
_NW = _NC * _NS
_C = 32


def _gather_call(S_batch, S_seq, D, G):
    mesh = plsc.VectorSubcoreMesh(core_axis_name="c", subcore_axis_name="s")
    per_w = G * _C
    w_per_row = S_seq // per_w

    @functools.partial(
        pl.kernel,
        out_type=jax.ShapeDtypeStruct((S_batch, S_seq, D), jnp.float32),
        mesh=mesh,
        scratch_types=[
            pltpu.VMEM((per_w,), jnp.int32),
            pltpu.VMEM((_C, D), jnp.float32),
            pltpu.VMEM((_C, D), jnp.float32),
            pltpu.VMEM((_C, D), jnp.float32),
            pltpu.SemaphoreType.DMA,
            pltpu.SemaphoreType.DMA,
        ],
    )
    def k(idx_hbm, table_hbm, out_hbm, idx_v, rows0, rows1, rows2, gsem, ssem):
        wid = lax.axis_index("s") * _NC + lax.axis_index("c")
        b = wid // w_per_row
        col = (wid % w_per_row) * per_w
        pltpu.sync_copy(idx_hbm.at[b, pl.ds(col, per_w)], idx_v)
        bufs = (rows0, rows1, rows2)
        nb = len(bufs)

        gathers = [None] * G
        scats = [None] * G
        for g in range(nb - 1):
            gathers[g] = pltpu.async_copy(
                table_hbm.at[idx_v.at[pl.ds(g * _C, _C)]], bufs[g], gsem)
        for g in range(G):
            gathers[g].wait()
            if g + nb - 1 < G:
                if g >= 1:
                    scats[g - 1].wait()
                gathers[g + nb - 1] = pltpu.async_copy(
                    table_hbm.at[idx_v.at[pl.ds((g + nb - 1) * _C, _C)]],
                    bufs[(g + nb - 1) % nb], gsem)
            scats[g] = pltpu.async_copy(
                bufs[g % nb], out_hbm.at[b, pl.ds(col + g * _C, _C)], ssem)
        for g in range(G - nb, G):
            if g >= 0:
                scats[g].wait()

    return k


def kernel(positions, pos_emb):
    S_batch, S_seq = positions.shape
    D = pos_emb.shape[1]
    G = positions.size // (_NW * _C)
    idx = positions.astype(jnp.int32)
    return _gather_call(S_batch, S_seq, D, G)(idx, pos_emb)
